# baseline (speedup 1.0000x reference)
import functools

import jax
import jax.numpy as jnp
from jax import lax
from jax.experimental import pallas as pl
from jax.experimental.pallas import tpu as pltpu
from jax.experimental.pallas import tpu_sc as plsc

VOCAB = 100000
EMBED_DIM = 300
DPAD = 384
MAX_NORM = 1.0
CTX = 200

NC = 2
NS = 16
NW = NC * NS
B_PAD = 256
B_PER_W = B_PAD // NW

R_SC = 2048
V_SC = NW * R_SC
T0 = 64000
BLK = 2000
CHUNK = 32
GRP = CHUNK // 8
NCHUNK = R_SC // CHUNK
NK = EMBED_DIM // 16


def _sc_gather_body(table_hbm, idx_hbm, out_hbm, idx_v, rows_v, sem):
    wid = lax.axis_index("s") * NC + lax.axis_index("c")
    base = wid * B_PER_W
    pltpu.sync_copy(idx_hbm.at[pl.ds(base, 16)], idx_v)
    rows = idx_v[...]
    for j in range(B_PER_W):
        pltpu.make_async_copy(
            table_hbm.at[pl.ds(rows[j], 1)], rows_v.at[pl.ds(j, 1)], sem
        ).start()
    for _ in range(B_PER_W):
        pltpu.make_async_copy(
            table_hbm.at[pl.ds(0, 1)], rows_v.at[pl.ds(0, 1)], sem
        ).wait()
    pltpu.sync_copy(rows_v, out_hbm.at[pl.ds(base, B_PER_W)])


@functools.lru_cache(maxsize=1)
def _make_sc_gather():
    return functools.partial(
        pl.kernel,
        mesh=plsc.VectorSubcoreMesh(core_axis_name="c", subcore_axis_name="s"),
        out_type=jax.ShapeDtypeStruct((B_PAD, EMBED_DIM), jnp.float32),
        scratch_types=[
            pltpu.VMEM((16,), jnp.int32),
            pltpu.VMEM((B_PER_W, EMBED_DIM), jnp.float32),
            pltpu.SemaphoreType.DMA,
        ],
    )(_sc_gather_body)


def _tc_avg_body(g_ref, out_ref):
    g = g_ref[...]
    ss = jnp.sum(g * g, axis=1, keepdims=True)
    scale = jnp.minimum(1.0, MAX_NORM / jnp.maximum(jnp.sqrt(ss), 1e-7))
    avg = jnp.sum(g * scale, axis=0, keepdims=True) * (1.0 / CTX)
    out_ref[...] = jnp.concatenate(
        [avg, jnp.zeros((1, DPAD - EMBED_DIM), jnp.float32)], axis=1
    )


def _tc_mv_body(avg_ref, w_ref, b_ref, out_ref):
    res = lax.dot_general(
        avg_ref[:, :EMBED_DIM], w_ref[...],
        (((1,), (1,)), ((), ())),
        preferred_element_type=jnp.float32,
    )
    out_ref[...] = (res + b_ref[...].reshape(1, BLK)).reshape(1, 1, BLK)


_GDN = lax.GatherDimensionNumbers(
    offset_dims=(), collapsed_slice_dims=(0,), start_index_map=(0,)
)


def _perm(x, idx):
    return lax.gather(
        x, idx[:, None], _GDN, (1,), mode=lax.GatherScatterMode.PROMISE_IN_BOUNDS
    )


def _sc_mv_body(w_hbm, avg_hbm, b_hbm, out_hbm,
                avg_v, buf0, buf1, b_v, out_v, sem0, sem1):
    wid = lax.axis_index("s") * NC + lax.axis_index("c")
    base = wid * R_SC
    pltpu.sync_copy(avg_hbm, avg_v)
    pltpu.sync_copy(b_hbm.at[pl.ds(base, R_SC)], b_v)

    iota = lax.iota(jnp.int32, 16)
    avg_vecs = [avg_v[pl.ds(16 * k, 16)] for k in range(NK)]
    tail_avg = jnp.where(iota < 16 * NK - (EMBED_DIM - 16),
                         0.0, avg_v[pl.ds(EMBED_DIM - 16, 16)])
    bfly = [iota ^ s for s in (1, 2, 4, 8)]

    bufs = (buf0, buf1)
    sems = (sem0, sem1)

    def issue(ch, bsel):
        row0 = base + ch * CHUNK
        for g in range(GRP):
            pltpu.make_async_copy(
                w_hbm.at[pl.ds(row0 + g * 8, 8)], bufs[bsel].at[g], sems[bsel]
            ).start()

    def drain(bsel):
        for g in range(GRP):
            pltpu.make_async_copy(
                w_hbm.at[pl.ds(0, 8)], bufs[bsel].at[g], sems[bsel]
            ).wait()

    def compute(ch, bsel):
        buf = bufs[bsel]
        for half in range(CHUNK // 16):
            out_vec = jnp.zeros((16,), jnp.float32)
            for r16 in range(16):
                r = half * 16 + r16
                g, rr = divmod(r, 8)
                acc = buf[g, rr, pl.ds(0, 16)] * avg_vecs[0]
                for k in range(1, NK):
                    acc = acc + buf[g, rr, pl.ds(16 * k, 16)] * avg_vecs[k]
                acc = acc + buf[g, rr, pl.ds(EMBED_DIM - 16, 16)] * tail_avg
                for p in bfly:
                    acc = acc + _perm(acc, p)
                out_vec = jnp.where(iota == r16, acc, out_vec)
            loc = ch * CHUNK + half * 16
            out_v[pl.ds(loc, 16)] = out_vec + b_v[pl.ds(loc, 16)]

    issue(0, 0)
    issue(1, 1)

    def pair(i, carry):
        ch0 = 2 * i
        drain(0)
        compute(ch0, 0)
        issue(ch0 + 2, 0)
        drain(1)
        compute(ch0 + 1, 1)
        issue(ch0 + 3, 1)
        return carry

    lax.fori_loop(0, NCHUNK // 2, pair, 0)
    drain(0)
    drain(1)
    pltpu.sync_copy(out_v, out_hbm.at[pl.ds(base, R_SC)])


@functools.lru_cache(maxsize=1)
def _make_sc_mv():
    return functools.partial(
        pl.kernel,
        mesh=plsc.VectorSubcoreMesh(core_axis_name="c", subcore_axis_name="s"),
        out_type=jax.ShapeDtypeStruct((V_SC,), jnp.float32),
        scratch_types=[
            pltpu.VMEM((DPAD,), jnp.float32),
            pltpu.VMEM((GRP, 8, EMBED_DIM), jnp.float32),
            pltpu.VMEM((GRP, 8, EMBED_DIM), jnp.float32),
            pltpu.VMEM((R_SC,), jnp.float32),
            pltpu.VMEM((R_SC,), jnp.float32),
            pltpu.SemaphoreType.DMA,
            pltpu.SemaphoreType.DMA,
        ],
    )(_sc_mv_body)


def kernel(inputs, emb_table, W, b):
    idx = jnp.zeros((B_PAD + 8,), jnp.int32).at[:CTX].set(inputs.astype(jnp.int32))
    gathered = _make_sc_gather()(emb_table, idx)

    avg = pl.pallas_call(
        _tc_avg_body,
        out_shape=jax.ShapeDtypeStruct((1, DPAD), jnp.float32),
    )(gathered[:CTX])

    sc_logits = _make_sc_mv()(W, avg.reshape(DPAD), b)

    nblk = (VOCAB - T0) // BLK
    off = T0 // BLK
    tc_out = pl.pallas_call(
        _tc_mv_body,
        grid=(nblk,),
        in_specs=[
            pl.BlockSpec((1, DPAD), lambda i: (0, 0)),
            pl.BlockSpec((BLK, EMBED_DIM), lambda i: (off + i, 0)),
            pl.BlockSpec((1, 1, BLK), lambda i: (off + i, 0, 0)),
        ],
        out_specs=pl.BlockSpec((1, 1, BLK), lambda i: (i, 0, 0)),
        out_shape=jax.ShapeDtypeStruct((nblk, 1, BLK), jnp.float32),
    )(avg, W, b.reshape(VOCAB // BLK, 1, BLK))

    logits = jnp.concatenate([sc_logits[:T0], tc_out.reshape(VOCAB - T0)])
    return logits.reshape(1, VOCAB)


# speedup vs baseline: 2.1604x; 2.1604x over previous
import functools

import jax
import jax.numpy as jnp
from jax import lax
from jax.experimental import pallas as pl
from jax.experimental.pallas import tpu as pltpu
from jax.experimental.pallas import tpu_sc as plsc

VOCAB = 100000
EMBED_DIM = 300
DPAD = 384
MAX_NORM = 1.0
CTX = 200

NC = 2
NS = 16
NW = NC * NS

V_SC = 40960
RV = V_SC // NW
CCH = 128
NCH = RV // CCH
BLKV = 2560
VMAIN_END = V_SC + ((VOCAB - V_SC) // BLKV) * BLKV
NTAIL = VOCAB - VMAIN_END


def _tc_avg_body(idx_ref, et_ref, wtail_ref, btail_ref, avg_ref, tail_ref,
                 buf0, buf1, acc_ref, sem0, sem1):
    lane = lax.broadcasted_iota(jnp.int32, (1, 128), 1)
    bufs = (buf0, buf1)
    sems = (sem0, sem1)

    def block_start(v):
        return pl.multiple_of(
            jnp.minimum((v // 128) * 128, VMAIN_END - 128), 128
        )

    def issue(j, s):
        v = idx_ref[j]
        pltpu.make_async_copy(
            et_ref.at[:, pl.ds(block_start(v), 128)], bufs[s], sems[s]
        ).start()

    acc_ref[...] = jnp.zeros((EMBED_DIM, 1), jnp.float32)
    issue(0, 0)
    issue(1, 1)

    tail_lane = lax.broadcasted_iota(jnp.int32, (1, NTAIL), 1)

    def process(j, s):
        pltpu.make_async_copy(
            et_ref.at[:, pl.ds(0, 128)], bufs[s], sems[s]
        ).wait()
        v = idx_ref[j]
        cc = v - block_start(v)
        arr = bufs[s][...]
        col_main = jnp.sum(arr * (lane == cc), axis=1, keepdims=True)
        col_tail = jnp.sum(
            wtail_ref[...] * (tail_lane == v - VMAIN_END), axis=1, keepdims=True
        )
        col = jnp.where(v >= VMAIN_END, col_tail, col_main)
        issue(j + 2, s)
        ss = jnp.sum(col * col)
        scale = jnp.minimum(1.0, MAX_NORM / jnp.maximum(jnp.sqrt(ss), 1e-7))
        acc_ref[...] = acc_ref[...] + col * scale

    def pair(i, carry):
        process(2 * i, 0)
        process(2 * i + 1, 1)
        return carry

    lax.fori_loop(0, CTX // 2, pair, 0)
    for s in (0, 1):
        pltpu.make_async_copy(
            et_ref.at[:, pl.ds(0, 128)], bufs[s], sems[s]
        ).wait()

    avg = acc_ref[...] * (1.0 / CTX)
    avg_ref[...] = jnp.concatenate(
        [avg, jnp.zeros((DPAD - EMBED_DIM, 1), jnp.float32)], axis=0
    )
    tail_ref[...] = (
        lax.dot_general(avg, wtail_ref[...], (((0,), (0,)), ((), ())),
                        preferred_element_type=jnp.float32)
        + btail_ref[...]
    )


def _tc_mv_body(avg_ref, w_ref, b_ref, out_ref):
    out_ref[...] = (
        lax.dot_general(
            avg_ref[: EMBED_DIM, :], w_ref[...],
            (((0,), (0,)), ((), ())),
            preferred_element_type=jnp.float32,
        )
        + b_ref[...]
    )


_GDN = lax.GatherDimensionNumbers(
    offset_dims=(), collapsed_slice_dims=(0,), start_index_map=(0,)
)


def _perm(x, idx):
    return lax.gather(
        x, idx[:, None], _GDN, (1,), mode=lax.GatherScatterMode.PROMISE_IN_BOUNDS
    )


def _sc_mv_body(wt_hbm, avg_hbm, b_hbm, out_hbm,
                avg_v, buf0, buf1, b_v, out_v, sem0, sem1):
    wid = lax.axis_index("s") * NC + lax.axis_index("c")
    base = wid * RV
    pltpu.sync_copy(avg_hbm, avg_v)
    pltpu.sync_copy(b_hbm.at[pl.ds(base, RV)], b_v)

    splats = [jnp.full((16,), r, jnp.int32) for r in range(16)]
    bufs = (buf0, buf1)
    sems = (sem0, sem1)

    def issue(ch, s):
        v0 = base + ch * CCH
        pltpu.make_async_copy(
            wt_hbm.at[:, pl.ds(v0, CCH)], bufs[s], sems[s]
        ).start()

    def drain(s):
        pltpu.make_async_copy(
            wt_hbm.at[:, pl.ds(0, CCH)], bufs[s], sems[s]
        ).wait()

    def compute(ch, s):
        buf = bufs[s]

        def qstep(q, accs):
            av = avg_v[pl.ds(q * 16, 16)]
            for r in range(16):
                c = q * 16 + r
                bc = _perm(av, splats[r])
                accs = tuple(
                    accs[l] + buf[c, pl.ds(16 * l, 16)] * bc for l in range(8)
                )
            return accs

        accs = tuple(jnp.zeros((16,), jnp.float32) for _ in range(8))
        accs = lax.fori_loop(0, EMBED_DIM // 16, qstep, accs)
        av18 = avg_v[pl.ds(288, 16)]
        for r in range(EMBED_DIM - 288):
            c = 288 + r
            bc = _perm(av18, splats[r])
            accs = tuple(
                accs[l] + buf[c, pl.ds(16 * l, 16)] * bc for l in range(8)
            )
        loc = ch * CCH
        for l in range(8):
            out_v[pl.ds(loc + 16 * l, 16)] = (
                accs[l] + b_v[pl.ds(loc + 16 * l, 16)]
            )

    issue(0, 0)
    issue(1, 1)

    def pair(i, carry):
        ch0 = 2 * i
        drain(0)
        compute(ch0, 0)
        issue(ch0 + 2, 0)
        drain(1)
        compute(ch0 + 1, 1)
        issue(ch0 + 3, 1)
        return carry

    lax.fori_loop(0, NCH // 2, pair, 0)
    drain(0)
    drain(1)
    pltpu.sync_copy(out_v, out_hbm.at[pl.ds(base, RV)])


@functools.lru_cache(maxsize=1)
def _make_sc_mv():
    return functools.partial(
        pl.kernel,
        mesh=plsc.VectorSubcoreMesh(core_axis_name="c", subcore_axis_name="s"),
        out_type=jax.ShapeDtypeStruct((V_SC,), jnp.float32),
        scratch_types=[
            pltpu.VMEM((DPAD,), jnp.float32),
            pltpu.VMEM((EMBED_DIM, CCH), jnp.float32),
            pltpu.VMEM((EMBED_DIM, CCH), jnp.float32),
            pltpu.VMEM((RV,), jnp.float32),
            pltpu.VMEM((RV,), jnp.float32),
            pltpu.SemaphoreType.DMA,
            pltpu.SemaphoreType.DMA,
        ],
    )(_sc_mv_body)


def kernel(inputs, emb_table, W, b):
    et = emb_table.T
    wt = W.T
    idx = jnp.zeros((CTX + 8,), jnp.int32).at[:CTX].set(inputs.astype(jnp.int32))
    wtail = lax.slice(wt, (0, VMAIN_END), (EMBED_DIM, VOCAB))
    btail = lax.slice(b, (VMAIN_END,), (VOCAB,)).reshape(1, NTAIL)

    avg, tail = pl.pallas_call(
        _tc_avg_body,
        in_specs=[
            pl.BlockSpec(memory_space=pltpu.MemorySpace.SMEM),
            pl.BlockSpec(memory_space=pltpu.MemorySpace.HBM),
            pl.BlockSpec((EMBED_DIM, NTAIL), lambda: (0, 0)),
            pl.BlockSpec((1, NTAIL), lambda: (0, 0)),
        ],
        out_shape=[
            jax.ShapeDtypeStruct((DPAD, 1), jnp.float32),
            jax.ShapeDtypeStruct((1, NTAIL), jnp.float32),
        ],
        scratch_shapes=[
            pltpu.VMEM((EMBED_DIM, 128), jnp.float32),
            pltpu.VMEM((EMBED_DIM, 128), jnp.float32),
            pltpu.VMEM((EMBED_DIM, 1), jnp.float32),
            pltpu.SemaphoreType.DMA,
            pltpu.SemaphoreType.DMA,
        ],
    )(idx, et, wtail, btail)

    sc_logits = _make_sc_mv()(wt, avg.reshape(DPAD), b)

    nblk = (VMAIN_END - V_SC) // BLKV
    off = V_SC // BLKV
    tc_main = pl.pallas_call(
        _tc_mv_body,
        grid=(nblk,),
        in_specs=[
            pl.BlockSpec((DPAD, 1), lambda i: (0, 0)),
            pl.BlockSpec((EMBED_DIM, BLKV), lambda i: (0, off + i)),
            pl.BlockSpec((1, BLKV), lambda i: (0, off + i)),
        ],
        out_specs=pl.BlockSpec((1, BLKV), lambda i: (0, i)),
        out_shape=jax.ShapeDtypeStruct((1, VMAIN_END - V_SC), jnp.float32),
    )(avg, wt, b.reshape(1, VOCAB))

    logits = jnp.concatenate(
        [sc_logits, tc_main.reshape(VMAIN_END - V_SC), tail.reshape(NTAIL)]
    )
    return logits.reshape(1, VOCAB)


# speedup vs baseline: 2.5199x; 1.1664x over previous
import functools

import jax
import jax.numpy as jnp
from jax import lax
from jax.experimental import pallas as pl
from jax.experimental.pallas import tpu as pltpu
from jax.experimental.pallas import tpu_sc as plsc

VOCAB = 100000
EMBED_DIM = 300
DPAD = 384
MAX_NORM = 1.0
CTX = 200

NC = 2
NS = 16
NW = NC * NS

V_SC = 40960
RV = V_SC // NW
CCH = 128
NCH = RV // CCH
BLKV = 2560
VMAIN_END = V_SC + ((VOCAB - V_SC) // BLKV) * BLKV
NTAIL = VOCAB - VMAIN_END


NBUF = 8


def _tc_avg_body(idx_ref, et_ref, wtail_ref, btail_ref, avg_ref, tail_ref,
                 *scratch):
    bufs = scratch[:NBUF]
    acc_ref = scratch[NBUF]
    sems = scratch[NBUF + 1:]
    lane = lax.broadcasted_iota(jnp.int32, (1, 128), 1)

    def block_start(v):
        return pl.multiple_of(
            jnp.minimum((v // 128) * 128, VMAIN_END - 128), 128
        )

    def issue(j, s):
        v = idx_ref[j]
        pltpu.make_async_copy(
            et_ref.at[:, pl.ds(block_start(v), 128)], bufs[s], sems[s]
        ).start()

    acc_ref[...] = jnp.zeros((EMBED_DIM, 1), jnp.float32)
    for s in range(NBUF):
        issue(s, s)

    tail_lane = lax.broadcasted_iota(jnp.int32, (1, NTAIL), 1)

    def process(j, s):
        pltpu.make_async_copy(
            et_ref.at[:, pl.ds(0, 128)], bufs[s], sems[s]
        ).wait()
        v = idx_ref[j]
        cc = v - block_start(v)
        arr = bufs[s][...]
        col_main = jnp.sum(arr * (lane == cc), axis=1, keepdims=True)
        col_tail = jnp.sum(
            wtail_ref[...] * (tail_lane == v - VMAIN_END), axis=1, keepdims=True
        )
        col = jnp.where(v >= VMAIN_END, col_tail, col_main)
        issue(j + NBUF, s)
        ss = jnp.sum(col * col)
        scale = jnp.minimum(1.0, MAX_NORM / jnp.maximum(jnp.sqrt(ss), 1e-7))
        acc_ref[...] = acc_ref[...] + col * scale

    def rnd(i, carry):
        for s in range(NBUF):
            process(NBUF * i + s, s)
        return carry

    lax.fori_loop(0, CTX // NBUF, rnd, 0)
    for s in range(NBUF):
        pltpu.make_async_copy(
            et_ref.at[:, pl.ds(0, 128)], bufs[s], sems[s]
        ).wait()

    avg = acc_ref[...] * (1.0 / CTX)
    avg_ref[...] = jnp.concatenate(
        [avg, jnp.zeros((DPAD - EMBED_DIM, 1), jnp.float32)], axis=0
    )
    tail_ref[...] = (
        lax.dot_general(avg, wtail_ref[...], (((0,), (0,)), ((), ())),
                        preferred_element_type=jnp.float32)
        + btail_ref[...]
    )


def _tc_mv_body(avg_ref, w_ref, b_ref, out_ref):
    out_ref[...] = (
        lax.dot_general(
            avg_ref[: EMBED_DIM, :], w_ref[...],
            (((0,), (0,)), ((), ())),
            preferred_element_type=jnp.float32,
        )
        + b_ref[...]
    )


_GDN = lax.GatherDimensionNumbers(
    offset_dims=(), collapsed_slice_dims=(0,), start_index_map=(0,)
)


def _perm(x, idx):
    return lax.gather(
        x, idx[:, None], _GDN, (1,), mode=lax.GatherScatterMode.PROMISE_IN_BOUNDS
    )


def _sc_mv_body(wt_hbm, avg_hbm, b_hbm, out_hbm,
                avg_v, buf0, buf1, b_v, out_v, sem0, sem1):
    wid = lax.axis_index("s") * NC + lax.axis_index("c")
    base = wid * RV
    pltpu.sync_copy(avg_hbm, avg_v)
    pltpu.sync_copy(b_hbm.at[pl.ds(base, RV)], b_v)

    splats = [jnp.full((16,), r, jnp.int32) for r in range(16)]
    bufs = (buf0, buf1)
    sems = (sem0, sem1)

    def issue(ch, s):
        v0 = base + ch * CCH
        pltpu.make_async_copy(
            wt_hbm.at[:, pl.ds(v0, CCH)], bufs[s], sems[s]
        ).start()

    def drain(s):
        pltpu.make_async_copy(
            wt_hbm.at[:, pl.ds(0, CCH)], bufs[s], sems[s]
        ).wait()

    def compute(ch, s):
        buf = bufs[s]

        def qstep(q, accs):
            av = avg_v[pl.ds(q * 16, 16)]
            for r in range(16):
                c = q * 16 + r
                bc = _perm(av, splats[r])
                accs = tuple(
                    accs[l] + buf[c, pl.ds(16 * l, 16)] * bc for l in range(8)
                )
            return accs

        accs = tuple(jnp.zeros((16,), jnp.float32) for _ in range(8))
        accs = lax.fori_loop(0, EMBED_DIM // 16, qstep, accs)
        av18 = avg_v[pl.ds(288, 16)]
        for r in range(EMBED_DIM - 288):
            c = 288 + r
            bc = _perm(av18, splats[r])
            accs = tuple(
                accs[l] + buf[c, pl.ds(16 * l, 16)] * bc for l in range(8)
            )
        loc = ch * CCH
        for l in range(8):
            out_v[pl.ds(loc + 16 * l, 16)] = (
                accs[l] + b_v[pl.ds(loc + 16 * l, 16)]
            )

    issue(0, 0)
    issue(1, 1)

    def pair(i, carry):
        ch0 = 2 * i
        drain(0)
        compute(ch0, 0)
        issue(ch0 + 2, 0)
        drain(1)
        compute(ch0 + 1, 1)
        issue(ch0 + 3, 1)
        return carry

    lax.fori_loop(0, NCH // 2, pair, 0)
    drain(0)
    drain(1)
    pltpu.sync_copy(out_v, out_hbm.at[pl.ds(base, RV)])


@functools.lru_cache(maxsize=1)
def _make_sc_mv():
    return functools.partial(
        pl.kernel,
        mesh=plsc.VectorSubcoreMesh(core_axis_name="c", subcore_axis_name="s"),
        out_type=jax.ShapeDtypeStruct((V_SC,), jnp.float32),
        scratch_types=[
            pltpu.VMEM((DPAD,), jnp.float32),
            pltpu.VMEM((EMBED_DIM, CCH), jnp.float32),
            pltpu.VMEM((EMBED_DIM, CCH), jnp.float32),
            pltpu.VMEM((RV,), jnp.float32),
            pltpu.VMEM((RV,), jnp.float32),
            pltpu.SemaphoreType.DMA,
            pltpu.SemaphoreType.DMA,
        ],
    )(_sc_mv_body)


def kernel(inputs, emb_table, W, b):
    et = emb_table.T
    wt = W.T
    idx = jnp.zeros((CTX + NBUF,), jnp.int32).at[:CTX].set(
        inputs.astype(jnp.int32)
    )
    wtail = lax.slice(wt, (0, VMAIN_END), (EMBED_DIM, VOCAB))
    btail = lax.slice(b, (VMAIN_END,), (VOCAB,)).reshape(1, NTAIL)

    avg, tail = pl.pallas_call(
        _tc_avg_body,
        in_specs=[
            pl.BlockSpec(memory_space=pltpu.MemorySpace.SMEM),
            pl.BlockSpec(memory_space=pltpu.MemorySpace.HBM),
            pl.BlockSpec((EMBED_DIM, NTAIL), lambda: (0, 0)),
            pl.BlockSpec((1, NTAIL), lambda: (0, 0)),
        ],
        out_shape=[
            jax.ShapeDtypeStruct((DPAD, 1), jnp.float32),
            jax.ShapeDtypeStruct((1, NTAIL), jnp.float32),
        ],
        scratch_shapes=(
            [pltpu.VMEM((EMBED_DIM, 128), jnp.float32)] * NBUF
            + [pltpu.VMEM((EMBED_DIM, 1), jnp.float32)]
            + [pltpu.SemaphoreType.DMA] * NBUF
        ),
    )(idx, et, wtail, btail)

    sc_logits = _make_sc_mv()(wt, avg.reshape(DPAD), b)

    nblk = (VMAIN_END - V_SC) // BLKV
    off = V_SC // BLKV
    tc_main = pl.pallas_call(
        _tc_mv_body,
        grid=(nblk,),
        in_specs=[
            pl.BlockSpec((DPAD, 1), lambda i: (0, 0)),
            pl.BlockSpec((EMBED_DIM, BLKV), lambda i: (0, off + i)),
            pl.BlockSpec((1, BLKV), lambda i: (0, off + i)),
        ],
        out_specs=pl.BlockSpec((1, BLKV), lambda i: (0, i)),
        out_shape=jax.ShapeDtypeStruct((1, VMAIN_END - V_SC), jnp.float32),
    )(avg, wt, b.reshape(1, VOCAB))

    logits = jnp.concatenate(
        [sc_logits, tc_main.reshape(VMAIN_END - V_SC), tail.reshape(NTAIL)]
    )
    return logits.reshape(1, VOCAB)


# speedup vs baseline: 2.5217x; 1.0007x over previous
import functools

import jax
import jax.numpy as jnp
from jax import lax
from jax.experimental import pallas as pl
from jax.experimental.pallas import tpu as pltpu
from jax.experimental.pallas import tpu_sc as plsc

VOCAB = 100000
EMBED_DIM = 300
DPAD = 384
MAX_NORM = 1.0
CTX = 200

NC = 2
NS = 16
NW = NC * NS

V_SC = 40960
RV = V_SC // NW
CCH = 128
NCH = RV // CCH
BLKV = 2560
VMAIN_END = V_SC + ((VOCAB - V_SC) // BLKV) * BLKV
NTAIL = VOCAB - VMAIN_END


NBUF = 4


def _tc_avg_body(idx_ref, et_ref, wtail_ref, btail_ref, avg_ref, tail_ref,
                 *scratch):
    bufs = scratch[:NBUF]
    acc_ref = scratch[NBUF]
    sems = scratch[NBUF + 1:]
    lane = lax.broadcasted_iota(jnp.int32, (1, 128), 1)

    def block_start(v):
        return pl.multiple_of(
            jnp.minimum((v // 128) * 128, VMAIN_END - 128), 128
        )

    def issue(j, s):
        v = idx_ref[j]
        pltpu.make_async_copy(
            et_ref.at[:, pl.ds(block_start(v), 128)], bufs[s], sems[s]
        ).start()

    acc_ref[...] = jnp.zeros((EMBED_DIM, 1), jnp.float32)
    for s in range(NBUF):
        issue(s, s)

    tail_lane = lax.broadcasted_iota(jnp.int32, (1, NTAIL), 1)

    def process(j, s):
        pltpu.make_async_copy(
            et_ref.at[:, pl.ds(0, 128)], bufs[s], sems[s]
        ).wait()
        v = idx_ref[j]
        cc = v - block_start(v)
        arr = bufs[s][...]
        col_main = jnp.sum(arr * (lane == cc), axis=1, keepdims=True)
        col_tail = jnp.sum(
            wtail_ref[...] * (tail_lane == v - VMAIN_END), axis=1, keepdims=True
        )
        col = jnp.where(v >= VMAIN_END, col_tail, col_main)
        ss = jnp.sum(col * col)
        scale = jnp.minimum(1.0, MAX_NORM / jnp.maximum(jnp.sqrt(ss), 1e-7))
        acc_ref[...] = acc_ref[...] + col * scale
        issue(j + NBUF, s)

    def rnd(i, carry):
        for s in range(NBUF):
            process(NBUF * i + s, s)
        return carry

    lax.fori_loop(0, CTX // NBUF, rnd, 0)
    for s in range(NBUF):
        pltpu.make_async_copy(
            et_ref.at[:, pl.ds(0, 128)], bufs[s], sems[s]
        ).wait()

    avg = acc_ref[...] * (1.0 / CTX)
    avg_ref[...] = jnp.concatenate(
        [avg, jnp.zeros((DPAD - EMBED_DIM, 1), jnp.float32)], axis=0
    )
    tail_ref[...] = (
        lax.dot_general(avg, wtail_ref[...], (((0,), (0,)), ((), ())),
                        preferred_element_type=jnp.float32)
        + btail_ref[...]
    )


def _tc_mv_body(avg_ref, w_ref, b_ref, out_ref):
    out_ref[...] = (
        lax.dot_general(
            avg_ref[: EMBED_DIM, :], w_ref[...],
            (((0,), (0,)), ((), ())),
            preferred_element_type=jnp.float32,
        )
        + b_ref[...]
    )


_GDN = lax.GatherDimensionNumbers(
    offset_dims=(), collapsed_slice_dims=(0,), start_index_map=(0,)
)


def _perm(x, idx):
    return lax.gather(
        x, idx[:, None], _GDN, (1,), mode=lax.GatherScatterMode.PROMISE_IN_BOUNDS
    )


def _sc_mv_body(wt_hbm, avg_hbm, b_hbm, out_hbm,
                avg_v, buf0, buf1, b_v, out_v, sem0, sem1):
    wid = lax.axis_index("s") * NC + lax.axis_index("c")
    base = wid * RV
    pltpu.sync_copy(avg_hbm, avg_v)
    pltpu.sync_copy(b_hbm.at[pl.ds(base, RV)], b_v)

    splats = [jnp.full((16,), r, jnp.int32) for r in range(16)]
    bufs = (buf0, buf1)
    sems = (sem0, sem1)

    def issue(ch, s):
        v0 = base + ch * CCH
        pltpu.make_async_copy(
            wt_hbm.at[:, pl.ds(v0, CCH)], bufs[s], sems[s]
        ).start()

    def drain(s):
        pltpu.make_async_copy(
            wt_hbm.at[:, pl.ds(0, CCH)], bufs[s], sems[s]
        ).wait()

    def compute(ch, s):
        buf = bufs[s]

        def qstep(q, accs):
            av = avg_v[pl.ds(q * 16, 16)]
            for r in range(16):
                c = q * 16 + r
                bc = _perm(av, splats[r])
                accs = tuple(
                    accs[l] + buf[c, pl.ds(16 * l, 16)] * bc for l in range(8)
                )
            return accs

        accs = tuple(jnp.zeros((16,), jnp.float32) for _ in range(8))
        accs = lax.fori_loop(0, EMBED_DIM // 16, qstep, accs)
        av18 = avg_v[pl.ds(288, 16)]
        for r in range(EMBED_DIM - 288):
            c = 288 + r
            bc = _perm(av18, splats[r])
            accs = tuple(
                accs[l] + buf[c, pl.ds(16 * l, 16)] * bc for l in range(8)
            )
        loc = ch * CCH
        for l in range(8):
            out_v[pl.ds(loc + 16 * l, 16)] = (
                accs[l] + b_v[pl.ds(loc + 16 * l, 16)]
            )

    issue(0, 0)
    issue(1, 1)

    def pair(i, carry):
        ch0 = 2 * i
        drain(0)
        compute(ch0, 0)
        issue(ch0 + 2, 0)
        drain(1)
        compute(ch0 + 1, 1)
        issue(ch0 + 3, 1)
        return carry

    lax.fori_loop(0, NCH // 2, pair, 0)
    drain(0)
    drain(1)
    pltpu.sync_copy(out_v, out_hbm.at[pl.ds(base, RV)])


@functools.lru_cache(maxsize=1)
def _make_sc_mv():
    return functools.partial(
        pl.kernel,
        mesh=plsc.VectorSubcoreMesh(core_axis_name="c", subcore_axis_name="s"),
        out_type=jax.ShapeDtypeStruct((V_SC,), jnp.float32),
        scratch_types=[
            pltpu.VMEM((DPAD,), jnp.float32),
            pltpu.VMEM((EMBED_DIM, CCH), jnp.float32),
            pltpu.VMEM((EMBED_DIM, CCH), jnp.float32),
            pltpu.VMEM((RV,), jnp.float32),
            pltpu.VMEM((RV,), jnp.float32),
            pltpu.SemaphoreType.DMA,
            pltpu.SemaphoreType.DMA,
        ],
    )(_sc_mv_body)


def kernel(inputs, emb_table, W, b):
    et = emb_table.T
    wt = W.T
    idx = jnp.zeros((CTX + NBUF,), jnp.int32).at[:CTX].set(
        inputs.astype(jnp.int32)
    )
    wtail = lax.slice(wt, (0, VMAIN_END), (EMBED_DIM, VOCAB))
    btail = lax.slice(b, (VMAIN_END,), (VOCAB,)).reshape(1, NTAIL)

    avg, tail = pl.pallas_call(
        _tc_avg_body,
        in_specs=[
            pl.BlockSpec(memory_space=pltpu.MemorySpace.SMEM),
            pl.BlockSpec(memory_space=pltpu.MemorySpace.HBM),
            pl.BlockSpec((EMBED_DIM, NTAIL), lambda: (0, 0)),
            pl.BlockSpec((1, NTAIL), lambda: (0, 0)),
        ],
        out_shape=[
            jax.ShapeDtypeStruct((DPAD, 1), jnp.float32),
            jax.ShapeDtypeStruct((1, NTAIL), jnp.float32),
        ],
        scratch_shapes=(
            [pltpu.VMEM((EMBED_DIM, 128), jnp.float32)] * NBUF
            + [pltpu.VMEM((EMBED_DIM, 1), jnp.float32)]
            + [pltpu.SemaphoreType.DMA] * NBUF
        ),
    )(idx, et, wtail, btail)

    sc_logits = _make_sc_mv()(wt, avg.reshape(DPAD), b)

    nblk = (VMAIN_END - V_SC) // BLKV
    off = V_SC // BLKV
    tc_main = pl.pallas_call(
        _tc_mv_body,
        grid=(nblk,),
        in_specs=[
            pl.BlockSpec((DPAD, 1), lambda i: (0, 0)),
            pl.BlockSpec((EMBED_DIM, BLKV), lambda i: (0, off + i)),
            pl.BlockSpec((1, BLKV), lambda i: (0, off + i)),
        ],
        out_specs=pl.BlockSpec((1, BLKV), lambda i: (0, i)),
        out_shape=jax.ShapeDtypeStruct((1, VMAIN_END - V_SC), jnp.float32),
    )(avg, wt, b.reshape(1, VOCAB))

    logits = jnp.concatenate(
        [sc_logits, tc_main.reshape(VMAIN_END - V_SC), tail.reshape(NTAIL)]
    )
    return logits.reshape(1, VOCAB)
